# baseline (device time: 68901 ns/iter reference)
import jax
import jax.numpy as jnp
from jax import lax
from jax.experimental import pallas as pl
from jax.experimental.pallas import tpu as pltpu

B, S, H, D = 2, 256, 8, 64
SG = 2 * S
SCALE = D ** -0.5


def kernel(Q, K, V):
    def body(q_ref, k_ref, v_ref, o_ref, kf_ref, vf_ref, send_sems, recv_sems):
        my_x = lax.axis_index("x")
        my_y = lax.axis_index("y")
        nbr = (my_x, 1 - my_y)

        barrier_sem = pltpu.get_barrier_semaphore()
        pl.semaphore_signal(
            barrier_sem, inc=1, device_id=nbr,
            device_id_type=pl.DeviceIdType.MESH,
        )
        pl.semaphore_wait(barrier_sem, 1)

        rdma_k = pltpu.make_async_remote_copy(
            src_ref=k_ref,
            dst_ref=kf_ref.at[:, pl.ds(my_y * S, S), :, :],
            send_sem=send_sems.at[0],
            recv_sem=recv_sems.at[0],
            device_id=nbr,
            device_id_type=pl.DeviceIdType.MESH,
        )
        rdma_v = pltpu.make_async_remote_copy(
            src_ref=v_ref,
            dst_ref=vf_ref.at[:, pl.ds(my_y * S, S), :, :],
            send_sem=send_sems.at[1],
            recv_sem=recv_sems.at[1],
            device_id=nbr,
            device_id_type=pl.DeviceIdType.MESH,
        )
        rdma_k.start()
        rdma_v.start()

        kf_ref[:, pl.ds(my_y * S, S), :, :] = k_ref[...]
        vf_ref[:, pl.ds(my_y * S, S), :, :] = v_ref[...]

        rdma_k.wait()
        rdma_v.wait()

        for b in range(B):
            for h in range(H):
                q = q_ref[b, :, h, :].astype(jnp.bfloat16)
                k = kf_ref[b, :, h, :].astype(jnp.bfloat16)
                v = vf_ref[b, :, h, :].astype(jnp.bfloat16)
                s = lax.dot_general(
                    q, k, (((1,), (1,)), ((), ())),
                    preferred_element_type=jnp.float32,
                ) * SCALE
                p = jnp.exp(s)
                l = jnp.sum(p, axis=-1, keepdims=True)
                o = lax.dot_general(
                    p.astype(jnp.bfloat16), v, (((1,), (0,)), ((), ())),
                    preferred_element_type=jnp.float32,
                )
                o_ref[b, :, h, :] = o * (1.0 / l)

    return pl.pallas_call(
        body,
        out_shape=jax.ShapeDtypeStruct((B, S, H, D), jnp.float32),
        in_specs=[
            pl.BlockSpec(memory_space=pltpu.VMEM),
            pl.BlockSpec(memory_space=pltpu.VMEM),
            pl.BlockSpec(memory_space=pltpu.VMEM),
        ],
        out_specs=pl.BlockSpec(memory_space=pltpu.VMEM),
        scratch_shapes=[
            pltpu.VMEM((B, SG, H, D), jnp.float32),
            pltpu.VMEM((B, SG, H, D), jnp.float32),
            pltpu.SemaphoreType.DMA((2,)),
            pltpu.SemaphoreType.DMA((2,)),
        ],
        compiler_params=pltpu.CompilerParams(collective_id=0),
    )(Q, K, V)


# device time: 58193 ns/iter; 1.1840x vs baseline; 1.1840x over previous
import jax
import jax.numpy as jnp
from jax import lax
from jax.experimental import pallas as pl
from jax.experimental.pallas import tpu as pltpu

B, S, H, D = 2, 256, 8, 64
SG = 2 * S
SCALE = D ** -0.5


def kernel(Q, K, V):
    Qt = jnp.transpose(Q, (0, 2, 1, 3))
    Kt = jnp.transpose(K, (0, 2, 1, 3))
    Vt = jnp.transpose(V, (0, 2, 1, 3))

    def body(q_ref, k_ref, v_ref, o_ref, kf_ref, vf_ref, send_sems, recv_sems):
        my_x = lax.axis_index("x")
        my_y = lax.axis_index("y")
        nbr = (my_x, 1 - my_y)

        barrier_sem = pltpu.get_barrier_semaphore()
        pl.semaphore_signal(
            barrier_sem, inc=1, device_id=nbr,
            device_id_type=pl.DeviceIdType.MESH,
        )
        pl.semaphore_wait(barrier_sem, 1)

        rdma_k = pltpu.make_async_remote_copy(
            src_ref=k_ref,
            dst_ref=kf_ref.at[:, :, pl.ds(my_y * S, S), :],
            send_sem=send_sems.at[0],
            recv_sem=recv_sems.at[0],
            device_id=nbr,
            device_id_type=pl.DeviceIdType.MESH,
        )
        rdma_v = pltpu.make_async_remote_copy(
            src_ref=v_ref,
            dst_ref=vf_ref.at[:, :, pl.ds(my_y * S, S), :],
            send_sem=send_sems.at[1],
            recv_sem=recv_sems.at[1],
            device_id=nbr,
            device_id_type=pl.DeviceIdType.MESH,
        )
        rdma_k.start()
        rdma_v.start()

        kf_ref[:, :, pl.ds(my_y * S, S), :] = k_ref[...]
        vf_ref[:, :, pl.ds(my_y * S, S), :] = v_ref[...]

        rdma_k.wait()
        rdma_v.wait()

        for b in range(B):
            for h in range(H):
                q = q_ref[b, h].astype(jnp.bfloat16)
                k = kf_ref[b, h].astype(jnp.bfloat16)
                v = vf_ref[b, h].astype(jnp.bfloat16)
                s = lax.dot_general(
                    q, k, (((1,), (1,)), ((), ())),
                    preferred_element_type=jnp.float32,
                ) * SCALE
                p = jnp.exp(s)
                l = jnp.sum(p, axis=-1, keepdims=True)
                o = lax.dot_general(
                    p.astype(jnp.bfloat16), v, (((1,), (0,)), ((), ())),
                    preferred_element_type=jnp.float32,
                )
                o_ref[b, h] = o * (1.0 / l)

    out_t = pl.pallas_call(
        body,
        out_shape=jax.ShapeDtypeStruct((B, H, S, D), jnp.float32),
        in_specs=[
            pl.BlockSpec(memory_space=pltpu.VMEM),
            pl.BlockSpec(memory_space=pltpu.VMEM),
            pl.BlockSpec(memory_space=pltpu.VMEM),
        ],
        out_specs=pl.BlockSpec(memory_space=pltpu.VMEM),
        scratch_shapes=[
            pltpu.VMEM((B, H, SG, D), jnp.float32),
            pltpu.VMEM((B, H, SG, D), jnp.float32),
            pltpu.SemaphoreType.DMA((2,)),
            pltpu.SemaphoreType.DMA((2,)),
        ],
        compiler_params=pltpu.CompilerParams(collective_id=0),
    )(Qt, Kt, Vt)

    return jnp.transpose(out_t, (0, 2, 1, 3))


# device time: 35569 ns/iter; 1.9371x vs baseline; 1.6361x over previous
import jax
import jax.numpy as jnp
from jax import lax
from jax.experimental import pallas as pl
from jax.experimental.pallas import tpu as pltpu

B, S, H, D = 2, 256, 8, 64
HALF = S // 2
SCALE = D ** -0.5


def kernel(Q, K, V):
    Qt = jnp.transpose(Q, (0, 2, 1, 3))
    Kt = jnp.transpose(K, (0, 2, 1, 3))
    Vt = jnp.transpose(V, (0, 2, 1, 3))

    def body(q_ref, k_ref, v_ref, o_ref, snd, rcv_y, rcv_x, l_ref, sems):
        my_x = lax.axis_index("x")
        my_y = lax.axis_index("y")
        nbr_y = (my_x, 1 - my_y)
        nbr_x = (1 - my_x, my_y)

        barrier_sem = pltpu.get_barrier_semaphore()
        for nbr in (nbr_y, nbr_x):
            pl.semaphore_signal(
                barrier_sem, inc=1, device_id=nbr,
                device_id_type=pl.DeviceIdType.MESH,
            )
        pl.semaphore_wait(barrier_sem, 2)

        snd[0] = k_ref[:, :, pl.ds(my_x * HALF, HALF), :].astype(jnp.bfloat16)
        snd[1] = v_ref[:, :, pl.ds(my_x * HALF, HALF), :].astype(jnp.bfloat16)

        rdma_y = pltpu.make_async_remote_copy(
            src_ref=snd, dst_ref=rcv_y,
            send_sem=sems.at[0], recv_sem=sems.at[1],
            device_id=nbr_y, device_id_type=pl.DeviceIdType.MESH,
        )
        rdma_y.start()

        def block(b, h, k_blk, v_blk, acc):
            q = q_ref[b, h].astype(jnp.bfloat16)
            s = lax.dot_general(
                q, k_blk, (((1,), (1,)), ((), ())),
                preferred_element_type=jnp.float32,
            ) * SCALE
            p = jnp.exp(s)
            l = jnp.sum(p, axis=-1, keepdims=True)
            o = lax.dot_general(
                p.astype(jnp.bfloat16), v_blk, (((1,), (0,)), ((), ())),
                preferred_element_type=jnp.float32,
            )
            return o, l

        for b in range(B):
            for h in range(H):
                o, l = block(b, h, k_ref[b, h].astype(jnp.bfloat16),
                             v_ref[b, h].astype(jnp.bfloat16), None)
                o_ref[b, h] = o
                l_ref[b, h] = l

        rdma_y.wait()

        rdma_x = pltpu.make_async_remote_copy(
            src_ref=rcv_y, dst_ref=rcv_x,
            send_sem=sems.at[2], recv_sem=sems.at[3],
            device_id=nbr_x, device_id_type=pl.DeviceIdType.MESH,
        )
        rdma_x.start()

        for b in range(B):
            for h in range(H):
                o, l = block(b, h, rcv_y[0, b, h], rcv_y[1, b, h], None)
                o_ref[b, h] += o
                l_ref[b, h] += l

        rdma_x.wait()

        for b in range(B):
            for h in range(H):
                o, l = block(b, h, rcv_x[0, b, h], rcv_x[1, b, h], None)
                o_ref[b, h] = (o_ref[b, h] + o) * (1.0 / (l_ref[b, h] + l))

    out_t = pl.pallas_call(
        body,
        out_shape=jax.ShapeDtypeStruct((B, H, S, D), jnp.float32),
        in_specs=[
            pl.BlockSpec(memory_space=pltpu.VMEM),
            pl.BlockSpec(memory_space=pltpu.VMEM),
            pl.BlockSpec(memory_space=pltpu.VMEM),
        ],
        out_specs=pl.BlockSpec(memory_space=pltpu.VMEM),
        scratch_shapes=[
            pltpu.VMEM((2, B, H, HALF, D), jnp.bfloat16),
            pltpu.VMEM((2, B, H, HALF, D), jnp.bfloat16),
            pltpu.VMEM((2, B, H, HALF, D), jnp.bfloat16),
            pltpu.VMEM((B, H, S, 1), jnp.float32),
            pltpu.SemaphoreType.DMA((4,)),
        ],
        compiler_params=pltpu.CompilerParams(collective_id=0),
    )(Qt, Kt, Vt)

    return jnp.transpose(out_t, (0, 2, 1, 3))


# device time: 27291 ns/iter; 2.5247x vs baseline; 1.3033x over previous
import jax
import jax.numpy as jnp
from jax import lax
from jax.experimental import pallas as pl
from jax.experimental.pallas import tpu as pltpu

B, S, H, D = 2, 256, 8, 64
HALF = S // 2
SCALE = D ** -0.5


def kernel(Q, K, V):
    Qt = jnp.transpose(Q, (0, 2, 1, 3))
    Kt = jnp.transpose(K, (0, 2, 1, 3))
    Vt = jnp.transpose(V, (0, 2, 1, 3))

    def body(q_ref, k_ref, v_ref, o_ref, snd, rcv_y, rcv_x, l_ref,
             sems_ys, sems_yr, sems_xs, sems_xr):
        my_x = lax.axis_index("x")
        my_y = lax.axis_index("y")
        nbr_y = (my_x, 1 - my_y)
        nbr_x = (1 - my_x, my_y)

        barrier_sem = pltpu.get_barrier_semaphore()
        for nbr in (nbr_y, nbr_x):
            pl.semaphore_signal(
                barrier_sem, inc=1, device_id=nbr,
                device_id_type=pl.DeviceIdType.MESH,
            )
        pl.semaphore_wait(barrier_sem, 2)

        snd[0] = k_ref[:, :, pl.ds(my_x * HALF, HALF), :].astype(jnp.bfloat16)
        snd[1] = v_ref[:, :, pl.ds(my_x * HALF, HALF), :].astype(jnp.bfloat16)

        QTR = HALF // 2

        def chunk_copy(src, dst, c, send_sem, recv_sem, dev):
            t, sub = divmod(c, 2)
            sl = (t, slice(None), slice(None), pl.ds(sub * QTR, QTR))
            return pltpu.make_async_remote_copy(
                src_ref=src.at[sl], dst_ref=dst.at[sl],
                send_sem=send_sem, recv_sem=recv_sem,
                device_id=dev, device_id_type=pl.DeviceIdType.MESH,
            )

        rdma_y = [
            chunk_copy(snd, rcv_y, c, sems_ys.at[c], sems_yr.at[c], nbr_y)
            for c in range(4)
        ]
        for r in rdma_y:
            r.start()

        def block(b, h, k_blk, v_blk, acc):
            q = q_ref[b, h].astype(jnp.bfloat16)
            s = lax.dot_general(
                q, k_blk, (((1,), (1,)), ((), ())),
                preferred_element_type=jnp.float32,
            ) * SCALE
            p = jnp.exp(s)
            l = jnp.sum(p, axis=-1, keepdims=True)
            o = lax.dot_general(
                p.astype(jnp.bfloat16), v_blk, (((1,), (0,)), ((), ())),
                preferred_element_type=jnp.float32,
            )
            return o, l

        for b in range(B):
            for h in range(H):
                o, l = block(b, h, k_ref[b, h].astype(jnp.bfloat16),
                             v_ref[b, h].astype(jnp.bfloat16), None)
                o_ref[b, h] = o
                l_ref[b, h] = l

        rdma_x = [
            chunk_copy(rcv_y, rcv_x, c, sems_xs.at[c], sems_xr.at[c], nbr_x)
            for c in range(4)
        ]
        for ry, rx in zip(rdma_y, rdma_x):
            ry.wait_recv()
            rx.start()

        for b in range(B):
            for h in range(H):
                o, l = block(b, h, rcv_y[0, b, h], rcv_y[1, b, h], None)
                o_ref[b, h] += o
                l_ref[b, h] += l

        for rx in rdma_x:
            rx.wait_recv()

        for b in range(B):
            for h in range(H):
                o, l = block(b, h, rcv_x[0, b, h], rcv_x[1, b, h], None)
                o_ref[b, h] = (o_ref[b, h] + o) * (1.0 / (l_ref[b, h] + l))

        for r in rdma_y:
            r.wait_send()
        for r in rdma_x:
            r.wait_send()

    out_t = pl.pallas_call(
        body,
        out_shape=jax.ShapeDtypeStruct((B, H, S, D), jnp.float32),
        in_specs=[
            pl.BlockSpec(memory_space=pltpu.VMEM),
            pl.BlockSpec(memory_space=pltpu.VMEM),
            pl.BlockSpec(memory_space=pltpu.VMEM),
        ],
        out_specs=pl.BlockSpec(memory_space=pltpu.VMEM),
        scratch_shapes=[
            pltpu.VMEM((2, B, H, HALF, D), jnp.bfloat16),
            pltpu.VMEM((2, B, H, HALF, D), jnp.bfloat16),
            pltpu.VMEM((2, B, H, HALF, D), jnp.bfloat16),
            pltpu.VMEM((B, H, S, 1), jnp.float32),
            pltpu.SemaphoreType.DMA((4,)),
            pltpu.SemaphoreType.DMA((4,)),
            pltpu.SemaphoreType.DMA((4,)),
            pltpu.SemaphoreType.DMA((4,)),
        ],
        compiler_params=pltpu.CompilerParams(collective_id=0),
    )(Qt, Kt, Vt)

    return jnp.transpose(out_t, (0, 2, 1, 3))


# device time: 20188 ns/iter; 3.4130x vs baseline; 1.3518x over previous
import jax
import jax.numpy as jnp
from jax import lax
from jax.experimental import pallas as pl
from jax.experimental.pallas import tpu as pltpu

B, S, H, D = 2, 256, 8, 64
HALF = S // 2
SCALE = D ** -0.5
COMM_DTYPE = jnp.float8_e4m3fn


def kernel(Q, K, V):
    Qt = jnp.transpose(Q, (0, 2, 1, 3))
    Kt = jnp.transpose(K, (0, 2, 1, 3))
    Vt = jnp.transpose(V, (0, 2, 1, 3))

    def body(q_ref, k_ref, v_ref, o_ref, snd, rcv_y, rcv_x, l_ref,
             sems_ys, sems_yr, sems_xs, sems_xr):
        my_x = lax.axis_index("x")
        my_y = lax.axis_index("y")
        nbr_y = (my_x, 1 - my_y)
        nbr_x = (1 - my_x, my_y)

        barrier_sem = pltpu.get_barrier_semaphore()
        for nbr in (nbr_y, nbr_x):
            pl.semaphore_signal(
                barrier_sem, inc=1, device_id=nbr,
                device_id_type=pl.DeviceIdType.MESH,
            )
        pl.semaphore_wait(barrier_sem, 2)

        snd[0] = k_ref[:, :, pl.ds(my_x * HALF, HALF), :].astype(COMM_DTYPE)
        snd[1] = v_ref[:, :, pl.ds(my_x * HALF, HALF), :].astype(COMM_DTYPE)

        QTR = HALF // 2

        def chunk_copy(src, dst, c, send_sem, recv_sem, dev):
            t, sub = divmod(c, 2)
            sl = (t, slice(None), slice(None), pl.ds(sub * QTR, QTR))
            return pltpu.make_async_remote_copy(
                src_ref=src.at[sl], dst_ref=dst.at[sl],
                send_sem=send_sem, recv_sem=recv_sem,
                device_id=dev, device_id_type=pl.DeviceIdType.MESH,
            )

        rdma_y = [
            chunk_copy(snd, rcv_y, c, sems_ys.at[c], sems_yr.at[c], nbr_y)
            for c in range(4)
        ]
        for r in rdma_y:
            r.start()

        def block(b, h, k_blk, v_blk, acc):
            q = q_ref[b, h].astype(jnp.bfloat16)
            s = lax.dot_general(
                q, k_blk, (((1,), (1,)), ((), ())),
                preferred_element_type=jnp.float32,
            ) * SCALE
            p = jnp.exp(s)
            l = jnp.sum(p, axis=-1, keepdims=True)
            o = lax.dot_general(
                p.astype(jnp.bfloat16), v_blk, (((1,), (0,)), ((), ())),
                preferred_element_type=jnp.float32,
            )
            return o, l

        for b in range(B):
            for h in range(H):
                o, l = block(b, h, k_ref[b, h].astype(jnp.bfloat16),
                             v_ref[b, h].astype(jnp.bfloat16), None)
                o_ref[b, h] = o
                l_ref[b, h] = l

        rdma_x = [
            chunk_copy(rcv_y, rcv_x, c, sems_xs.at[c], sems_xr.at[c], nbr_x)
            for c in range(4)
        ]
        for ry, rx in zip(rdma_y, rdma_x):
            ry.wait_recv()
            rx.start()

        for b in range(B):
            for h in range(H):
                o, l = block(b, h, rcv_y[0, b, h].astype(jnp.bfloat16),
                             rcv_y[1, b, h].astype(jnp.bfloat16), None)
                o_ref[b, h] += o
                l_ref[b, h] += l

        for rx in rdma_x:
            rx.wait_recv()

        for b in range(B):
            for h in range(H):
                o, l = block(b, h, rcv_x[0, b, h].astype(jnp.bfloat16),
                             rcv_x[1, b, h].astype(jnp.bfloat16), None)
                o_ref[b, h] = (o_ref[b, h] + o) * (1.0 / (l_ref[b, h] + l))

        for r in rdma_y:
            r.wait_send()
        for r in rdma_x:
            r.wait_send()

    out_t = pl.pallas_call(
        body,
        out_shape=jax.ShapeDtypeStruct((B, H, S, D), jnp.float32),
        in_specs=[
            pl.BlockSpec(memory_space=pltpu.VMEM),
            pl.BlockSpec(memory_space=pltpu.VMEM),
            pl.BlockSpec(memory_space=pltpu.VMEM),
        ],
        out_specs=pl.BlockSpec(memory_space=pltpu.VMEM),
        scratch_shapes=[
            pltpu.VMEM((2, B, H, HALF, D), COMM_DTYPE),
            pltpu.VMEM((2, B, H, HALF, D), COMM_DTYPE),
            pltpu.VMEM((2, B, H, HALF, D), COMM_DTYPE),
            pltpu.VMEM((B, H, S, 1), jnp.float32),
            pltpu.SemaphoreType.DMA((4,)),
            pltpu.SemaphoreType.DMA((4,)),
            pltpu.SemaphoreType.DMA((4,)),
            pltpu.SemaphoreType.DMA((4,)),
        ],
        compiler_params=pltpu.CompilerParams(collective_id=0),
    )(Qt, Kt, Vt)

    return jnp.transpose(out_t, (0, 2, 1, 3))
